# Initial kernel scaffold; baseline (speedup 1.0000x reference)
import jax
import jax.numpy as jnp
from jax.experimental import pallas as pl


def kernel(train_x1, train_x2, fm_first_w, fm_second_w, bias, W1, b1, g1, be1, W2, b2, g2, be2, W3, b3, g3, be3):
    raise NotImplementedError("write your pallas kernel here")


# speedup vs baseline: 1.2363x; 1.2363x over previous
import functools

import jax
import jax.numpy as jnp
from jax import lax
from jax.experimental import pallas as pl
from jax.experimental.pallas import tpu as pltpu
from jax.experimental.pallas import tpu_sc as plsc

B = 16384
FIELD = 26
EMB = 16
D = FIELD * EMB
NUMF = 13
EPS = 1e-5

NC, NS = 2, 16
NW = NC * NS
N_IDX = B * FIELD
PER_W = N_IDX // NW
K = 128
NSTREAM = PER_W // K

_sc_mesh = plsc.VectorSubcoreMesh(core_axis_name="c", subcore_axis_name="s")


@functools.partial(
    pl.kernel,
    out_type=(
        jax.ShapeDtypeStruct((N_IDX, EMB), jnp.float32),
        jax.ShapeDtypeStruct((N_IDX,), jnp.float32),
    ),
    mesh=_sc_mesh,
    compiler_params=pltpu.CompilerParams(use_tc_tiling_on_sc=False),
    scratch_types=[
        pltpu.VMEM((NSTREAM, K), jnp.int32),
        pltpu.VMEM((K, EMB), jnp.float32),
        pltpu.VMEM((K,), jnp.float32),
        pltpu.SemaphoreType.DMA,
        pltpu.SemaphoreType.DMA,
    ],
)
def _sc_gather(idx_hbm, second_hbm, first_hbm, emb_out, first_out,
               idx_v, rows_v, fst_v, sem_r, sem_f):
    wid = lax.axis_index("s") * NC + lax.axis_index("c")
    row0 = wid * NSTREAM
    base = wid * PER_W
    pltpu.sync_copy(idx_hbm.at[pl.ds(row0, NSTREAM)], idx_v)

    def body(j, carry):
        pltpu.async_copy(second_hbm.at[idx_v.at[j]], rows_v, sem_r).wait()
        pltpu.sync_copy(rows_v, emb_out.at[pl.ds(base + j * K, K)])
        pltpu.async_copy(first_hbm.at[idx_v.at[j]], fst_v, sem_f).wait()
        pltpu.sync_copy(fst_v, first_out.at[pl.ds(base + j * K, K)])
        return carry

    lax.fori_loop(0, NSTREAM, body, 0)


def _mlp_body(emb_ref, x2_ref, fst_ref, s_ref,
              w1a_ref, w1b_ref, b1_ref, g1_ref, be1_ref,
              w2_ref, b2_ref, g2_ref, be2_ref,
              w3_ref, b3_ref, g3_ref, be3_ref, bias_ref, out_ref):
    x = emb_ref[...]
    inv = 1.0 / jnp.sqrt(1.0 + EPS)
    sumvec = jnp.dot(x, s_ref[...], preferred_element_type=jnp.float32)
    fm2 = 0.5 * (jnp.sum(sumvec * sumvec, axis=1) - jnp.sum(x * x, axis=1))
    h = (jnp.dot(x, w1a_ref[...], preferred_element_type=jnp.float32)
         + jnp.dot(x2_ref[...], w1b_ref[...], preferred_element_type=jnp.float32)
         + b1_ref[...])
    h = jnp.maximum(h * (g1_ref[...] * inv) + be1_ref[...], 0.0)
    h = jnp.dot(h, w2_ref[...], preferred_element_type=jnp.float32) + b2_ref[...]
    h = jnp.maximum(h * (g2_ref[...] * inv) + be2_ref[...], 0.0)
    h = jnp.dot(h, w3_ref[...], preferred_element_type=jnp.float32) + b3_ref[...]
    h = jnp.maximum(h * (g3_ref[...] * inv) + be3_ref[...], 0.0)
    logit = (jnp.sum(fst_ref[...], axis=1) + fm2 + jnp.sum(h, axis=1)
             + bias_ref[0])
    out_ref[...] = jax.nn.sigmoid(logit)


def _mlp(emb, x2, fst, s, w1a, w1b, b1, g1, be1, w2, b2, g2, be2,
         w3, b3, g3, be3, bias):
    bs = 2048
    grid = (B // bs,)
    full = lambda shape: pl.BlockSpec(shape, lambda i: tuple(0 for _ in shape))
    return pl.pallas_call(
        _mlp_body,
        grid=grid,
        in_specs=[
            pl.BlockSpec((bs, D), lambda i: (i, 0)),
            pl.BlockSpec((bs, NUMF), lambda i: (i, 0)),
            pl.BlockSpec((bs, FIELD), lambda i: (i, 0)),
            full(s.shape), full(w1a.shape), full(w1b.shape), full(b1.shape),
            full(g1.shape), full(be1.shape), full(w2.shape), full(b2.shape),
            full(g2.shape), full(be2.shape), full(w3.shape), full(b3.shape),
            full(g3.shape), full(be3.shape), full(bias.shape),
        ],
        out_specs=pl.BlockSpec((bs,), lambda i: (i,)),
        out_shape=jax.ShapeDtypeStruct((B,), jnp.float32),
    )(emb, x2, fst, s, w1a, w1b, b1, g1, be1, w2, b2, g2, be2,
      w3, b3, g3, be3, bias)


def kernel(train_x1, train_x2, fm_first_w, fm_second_w, bias,
           W1, b1, g1, be1, W2, b2, g2, be2, W3, b3, g3, be3):
    idx = train_x1.reshape(N_IDX // K, K)
    emb_flat, fst_flat = _sc_gather(idx, fm_second_w, fm_first_w.reshape(-1))
    emb = emb_flat.reshape(B, D)
    fst = fst_flat.reshape(B, FIELD)
    s = jnp.tile(jnp.eye(EMB, dtype=jnp.float32), (FIELD, 1))
    return _mlp(emb, train_x2, fst, s, W1[:D], W1[D:], b1, g1, be1,
                W2, b2, g2, be2, W3, b3, g3, be3, bias)


# speedup vs baseline: 1.5281x; 1.2360x over previous
import functools

import jax
import jax.numpy as jnp
from jax import lax
from jax.experimental import pallas as pl
from jax.experimental.pallas import tpu as pltpu
from jax.experimental.pallas import tpu_sc as plsc

B = 16384
FIELD = 26
EMB = 16
D = FIELD * EMB
NUMF = 13
EPS = 1e-5

NC, NS = 2, 16
NW = NC * NS
N_IDX = B * FIELD
PER_W = N_IDX // NW
K = 128
NSTREAM = PER_W // K

NB = 4
MB = NB * K
NMACRO = NSTREAM // NB

_sc_mesh = plsc.VectorSubcoreMesh(core_axis_name="c", subcore_axis_name="s")


@functools.partial(
    pl.kernel,
    out_type=(
        jax.ShapeDtypeStruct((N_IDX, EMB), jnp.float32),
        jax.ShapeDtypeStruct((N_IDX,), jnp.float32),
    ),
    mesh=_sc_mesh,
    compiler_params=pltpu.CompilerParams(use_tc_tiling_on_sc=False),
    scratch_types=[
        pltpu.VMEM((NSTREAM, K), jnp.int32),
        pltpu.VMEM((2, MB, EMB), jnp.float32),
        pltpu.VMEM((2, MB), jnp.float32),
        pltpu.SemaphoreType.DMA,
        pltpu.SemaphoreType.DMA,
        pltpu.SemaphoreType.DMA,
        pltpu.SemaphoreType.DMA,
    ],
)
def _sc_gather(idx_hbm, second_hbm, first_hbm, emb_out, first_out,
               idx_v, rows2, fst2, rsem0, rsem1, fsem0, fsem1):
    wid = lax.axis_index("s") * NC + lax.axis_index("c")
    row0 = wid * NSTREAM
    base = wid * PER_W
    rsem = (rsem0, rsem1)
    fsem = (fsem0, fsem1)
    pltpu.sync_copy(idx_hbm.at[pl.ds(row0, NSTREAM)], idx_v)

    def fire(m, p):
        for b in range(NB):
            j = m * NB + b
            pltpu.async_copy(second_hbm.at[idx_v.at[j]],
                             rows2.at[p].at[pl.ds(b * K, K)], rsem[p])
            pltpu.async_copy(first_hbm.at[idx_v.at[j]],
                             fst2.at[p].at[pl.ds(b * K, K)], fsem[p])

    def drain(m, p):
        for b in range(NB):
            j = m * NB + b
            pltpu.make_async_copy(second_hbm.at[idx_v.at[j]],
                                  rows2.at[p].at[pl.ds(b * K, K)],
                                  rsem[p]).wait()
            pltpu.make_async_copy(first_hbm.at[idx_v.at[j]],
                                  fst2.at[p].at[pl.ds(b * K, K)],
                                  fsem[p]).wait()

    def write(m, p):
        pltpu.sync_copy(rows2.at[p], emb_out.at[pl.ds(base + m * MB, MB)])
        pltpu.sync_copy(fst2.at[p], first_out.at[pl.ds(base + m * MB, MB)])

    fire(0, 0)

    def body(i, carry):
        mm = 2 * i
        fire(mm + 1, 1)
        drain(mm, 0)
        write(mm, 0)

        @pl.when(i < NMACRO // 2 - 1)
        def _():
            fire(mm + 2, 0)

        drain(mm + 1, 1)
        write(mm + 1, 1)
        return carry

    lax.fori_loop(0, NMACRO // 2, body, 0)


def _mlp_body(emb_ref, x2_ref, fst_ref, s_ref,
              w1a_ref, w1b_ref, b1_ref, g1_ref, be1_ref,
              w2_ref, b2_ref, g2_ref, be2_ref,
              w3_ref, b3_ref, g3_ref, be3_ref, bias_ref, out_ref):
    x = emb_ref[...]
    inv = 1.0 / jnp.sqrt(1.0 + EPS)
    sumvec = jnp.dot(x, s_ref[...], preferred_element_type=jnp.float32)
    fm2 = 0.5 * (jnp.sum(sumvec * sumvec, axis=1) - jnp.sum(x * x, axis=1))
    h = (jnp.dot(x, w1a_ref[...], preferred_element_type=jnp.float32)
         + jnp.dot(x2_ref[...], w1b_ref[...], preferred_element_type=jnp.float32)
         + b1_ref[...])
    h = jnp.maximum(h * (g1_ref[...] * inv) + be1_ref[...], 0.0)
    h = jnp.dot(h, w2_ref[...], preferred_element_type=jnp.float32) + b2_ref[...]
    h = jnp.maximum(h * (g2_ref[...] * inv) + be2_ref[...], 0.0)
    h = jnp.dot(h, w3_ref[...], preferred_element_type=jnp.float32) + b3_ref[...]
    h = jnp.maximum(h * (g3_ref[...] * inv) + be3_ref[...], 0.0)
    logit = (jnp.sum(fst_ref[...], axis=1) + fm2 + jnp.sum(h, axis=1)
             + bias_ref[0])
    out_ref[...] = jax.nn.sigmoid(logit)


def _mlp(emb, x2, fst, s, w1a, w1b, b1, g1, be1, w2, b2, g2, be2,
         w3, b3, g3, be3, bias):
    bs = 2048
    grid = (B // bs,)
    full = lambda shape: pl.BlockSpec(shape, lambda i: tuple(0 for _ in shape))
    return pl.pallas_call(
        _mlp_body,
        grid=grid,
        in_specs=[
            pl.BlockSpec((bs, D), lambda i: (i, 0)),
            pl.BlockSpec((bs, NUMF), lambda i: (i, 0)),
            pl.BlockSpec((bs, FIELD), lambda i: (i, 0)),
            full(s.shape), full(w1a.shape), full(w1b.shape), full(b1.shape),
            full(g1.shape), full(be1.shape), full(w2.shape), full(b2.shape),
            full(g2.shape), full(be2.shape), full(w3.shape), full(b3.shape),
            full(g3.shape), full(be3.shape), full(bias.shape),
        ],
        out_specs=pl.BlockSpec((bs,), lambda i: (i,)),
        out_shape=jax.ShapeDtypeStruct((B,), jnp.float32),
    )(emb, x2, fst, s, w1a, w1b, b1, g1, be1, w2, b2, g2, be2,
      w3, b3, g3, be3, bias)


def kernel(train_x1, train_x2, fm_first_w, fm_second_w, bias,
           W1, b1, g1, be1, W2, b2, g2, be2, W3, b3, g3, be3):
    idx = train_x1.reshape(N_IDX // K, K)
    emb_flat, fst_flat = _sc_gather(idx, fm_second_w, fm_first_w.reshape(-1))
    emb = emb_flat.reshape(B, D)
    fst = fst_flat.reshape(B, FIELD)
    s = jnp.tile(jnp.eye(EMB, dtype=jnp.float32), (FIELD, 1))
    return _mlp(emb, train_x2, fst, s, W1[:D], W1[D:], b1, g1, be1,
                W2, b2, g2, be2, W3, b3, g3, be3, bias)


# speedup vs baseline: 1.6422x; 1.0747x over previous
import functools

import jax
import jax.numpy as jnp
from jax import lax
from jax.experimental import pallas as pl
from jax.experimental.pallas import tpu as pltpu
from jax.experimental.pallas import tpu_sc as plsc

B = 16384
FIELD = 26
EMB = 16
V = 26 * 40000
D = FIELD * EMB
NUMF = 13
EPS = 1e-5

NC, NS = 2, 16
NW = NC * NS
N_IDX = B * FIELD
PER_W = N_IDX // NW
K = 128
NSTREAM = PER_W // K

NB = 4
MB = NB * K
NMACRO = NSTREAM // NB
SAMP_W = B // NW

TCOL = V // 128
CTC = 13
CV = CTC * 128
NCHUNK = TCOL // CTC
NCHUNK_LO = NCHUNK // NW
NCHUNK_REM = NCHUNK - NCHUNK_LO * NW

_sc_mesh = plsc.VectorSubcoreMesh(core_axis_name="c", subcore_axis_name="s")


@functools.partial(
    pl.kernel,
    out_type=jax.ShapeDtypeStruct((V, EMB), jnp.float32),
    mesh=_sc_mesh,
    compiler_params=pltpu.CompilerParams(use_tc_tiling_on_sc=False,
                                        needs_layout_passes=False),
    scratch_types=[
        pltpu.VMEM((2, 2, CTC, 8, 128), jnp.float32),
        pltpu.VMEM((2, CV, EMB), jnp.float32),
        pltpu.SemaphoreType.DMA,
        pltpu.SemaphoreType.DMA,
        pltpu.SemaphoreType.DMA,
        pltpu.SemaphoreType.DMA,
    ],
)
def _sc_transpose(tq_hbm, t2_out, in4, out2,
                  isem0, isem1, osem0, osem1):
    wid = lax.axis_index("s") * NC + lax.axis_index("c")
    cnt = jnp.where(wid < NCHUNK_REM, NCHUNK_LO + 1, NCHUNK_LO)
    isem = (isem0, isem1)
    osem = (osem0, osem1)
    iota16 = lax.iota(jnp.int32, EMB)
    d_hi = iota16 // 8
    d_lo = iota16 % 8

    def fetch(i, p):
        c = wid + NW * i
        for tr in range(2):
            pltpu.async_copy(tq_hbm.at[tr].at[pl.ds(c * CTC, CTC)],
                             in4.at[p].at[tr], isem[p])

    def wait_fetch(i, p):
        c = wid + NW * i
        for tr in range(2):
            pltpu.make_async_copy(tq_hbm.at[tr].at[pl.ds(c * CTC, CTC)],
                                  in4.at[p].at[tr], isem[p]).wait()

    def wait_write(i, p):
        c = wid + NW * i
        pltpu.make_async_copy(out2.at[p], t2_out.at[pl.ds(c * CV, CV)],
                              osem[p]).wait()

    fetch(0, 0)

    def chunk_body(i, p):
        @pl.when(i + 1 < cnt)
        def _():
            fetch(i + 1, 1 - p)
        wait_fetch(i, p)
        src = in4.at[p]
        dst = out2.at[p]

        def interleave(v, carry):
            tc = jnp.full((EMB,), v // 128, jnp.int32)
            vv = jnp.full((EMB,), v % 128, jnp.int32)
            vec = plsc.load_gather(src, [d_hi, tc, d_lo, vv])
            dst[v] = vec
            return carry

        lax.fori_loop(0, CV, interleave, 0)
        c = wid + NW * i
        pltpu.async_copy(out2.at[p], t2_out.at[pl.ds(c * CV, CV)], osem[p])

    def body(j, carry):
        i = 2 * j

        @pl.when(i < cnt)
        def _():
            @pl.when(i >= 2)
            def _():
                wait_write(i - 2, 0)
            chunk_body(i, 0)

        @pl.when(i + 1 < cnt)
        def _():
            @pl.when(i >= 1)
            def _():
                wait_write(i - 1, 1)
            chunk_body(i + 1, 1)

        return carry

    lax.fori_loop(0, (NCHUNK_LO + 2) // 2, body, 0)

    hi = NCHUNK_LO + 1

    @pl.when(cnt == hi)
    def _():
        wait_write(hi - 2, (hi - 2) % 2)
        wait_write(hi - 1, (hi - 1) % 2)

    @pl.when(cnt == hi - 1)
    def _():
        wait_write(hi - 3, (hi - 3) % 2)
        wait_write(hi - 2, (hi - 2) % 2)


@functools.partial(
    pl.kernel,
    out_type=(
        jax.ShapeDtypeStruct((N_IDX, EMB), jnp.float32),
        jax.ShapeDtypeStruct((B,), jnp.float32),
    ),
    mesh=_sc_mesh,
    compiler_params=pltpu.CompilerParams(use_tc_tiling_on_sc=False,
                                        needs_layout_passes=False),
    scratch_types=[
        pltpu.VMEM((NSTREAM, K), jnp.int32),
        pltpu.VMEM((2, MB, EMB), jnp.float32),
        pltpu.VMEM((PER_W,), jnp.float32),
        pltpu.VMEM((SAMP_W,), jnp.float32),
        pltpu.SemaphoreType.DMA,
        pltpu.SemaphoreType.DMA,
        pltpu.SemaphoreType.DMA,
    ],
)
def _sc_gather(idx_hbm, second_hbm, first_hbm, emb_out, fsum_out,
               idx_v, rows2, fst_v, fsum_v, rsem0, rsem1, fsem):
    wid = lax.axis_index("s") * NC + lax.axis_index("c")
    row0 = wid * NSTREAM
    base = wid * PER_W
    rsem = (rsem0, rsem1)
    pltpu.sync_copy(idx_hbm.at[pl.ds(row0, NSTREAM)], idx_v)

    def fire(m, p):
        for b in range(NB):
            j = m * NB + b
            pltpu.async_copy(second_hbm.at[idx_v.at[j]],
                             rows2.at[p].at[pl.ds(b * K, K)], rsem[p])
            pltpu.async_copy(first_hbm.at[idx_v.at[j]],
                             fst_v.at[pl.ds((m * NB + b) * K, K)], fsem)

    def drain(m, p):
        for b in range(NB):
            j = m * NB + b
            pltpu.make_async_copy(second_hbm.at[idx_v.at[j]],
                                  rows2.at[p].at[pl.ds(b * K, K)],
                                  rsem[p]).wait()

    def write(m, p):
        pltpu.sync_copy(rows2.at[p], emb_out.at[pl.ds(base + m * MB, MB)])

    fire(0, 0)

    def body(i, carry):
        mm = 2 * i
        fire(mm + 1, 1)
        drain(mm, 0)
        write(mm, 0)

        @pl.when(i < NMACRO // 2 - 1)
        def _():
            fire(mm + 2, 0)

        drain(mm + 1, 1)
        write(mm + 1, 1)
        return carry

    lax.fori_loop(0, NMACRO // 2, body, 0)

    def fdrain(j, carry):
        pltpu.make_async_copy(first_hbm.at[idx_v.at[j]],
                              fst_v.at[pl.ds(j * K, K)], fsem).wait()
        return carry

    lax.fori_loop(0, NSTREAM, fdrain, 0)

    iota16 = lax.iota(jnp.int32, 16)

    def fsum_body(g, carry):
        acc = jnp.zeros((16,), jnp.float32)
        for f in range(FIELD):
            ridx = (g * 16 + iota16) * FIELD + f
            acc = acc + plsc.load_gather(fst_v, [ridx])
        fsum_v[pl.ds(g * 16, 16)] = acc
        return carry

    lax.fori_loop(0, SAMP_W // 16, fsum_body, 0)
    pltpu.sync_copy(fsum_v, fsum_out.at[pl.ds(wid * SAMP_W, SAMP_W)])


def _mlp_body(emb_ref, x2_ref, fsum_ref, s_ref,
              w1a_ref, w1b_ref, b1_ref, g1_ref, be1_ref,
              w2_ref, b2_ref, g2_ref, be2_ref,
              w3_ref, b3_ref, g3_ref, be3_ref, bias_ref, out_ref):
    x = emb_ref[...]
    inv = 1.0 / jnp.sqrt(1.0 + EPS)
    sumvec = jnp.dot(x, s_ref[...], preferred_element_type=jnp.float32)
    fm2 = 0.5 * (jnp.sum(sumvec * sumvec, axis=1) - jnp.sum(x * x, axis=1))
    h = (jnp.dot(x, w1a_ref[...], preferred_element_type=jnp.float32)
         + jnp.dot(x2_ref[...], w1b_ref[...], preferred_element_type=jnp.float32)
         + b1_ref[...])
    h = jnp.maximum(h * (g1_ref[...] * inv) + be1_ref[...], 0.0)
    h = jnp.dot(h, w2_ref[...], preferred_element_type=jnp.float32) + b2_ref[...]
    h = jnp.maximum(h * (g2_ref[...] * inv) + be2_ref[...], 0.0)
    h = jnp.dot(h, w3_ref[...], preferred_element_type=jnp.float32) + b3_ref[...]
    h = jnp.maximum(h * (g3_ref[...] * inv) + be3_ref[...], 0.0)
    logit = fsum_ref[...] + fm2 + jnp.sum(h, axis=1) + bias_ref[0]
    out_ref[...] = jax.nn.sigmoid(logit)


def _mlp(emb, x2, fsum, s, w1a, w1b, b1, g1, be1, w2, b2, g2, be2,
         w3, b3, g3, be3, bias):
    bs = 2048
    grid = (B // bs,)
    full = lambda shape: pl.BlockSpec(shape, lambda i: tuple(0 for _ in shape))
    return pl.pallas_call(
        _mlp_body,
        grid=grid,
        in_specs=[
            pl.BlockSpec((bs, D), lambda i: (i, 0)),
            pl.BlockSpec((bs, NUMF), lambda i: (i, 0)),
            pl.BlockSpec((bs,), lambda i: (i,)),
            full(s.shape), full(w1a.shape), full(w1b.shape), full(b1.shape),
            full(g1.shape), full(be1.shape), full(w2.shape), full(b2.shape),
            full(g2.shape), full(be2.shape), full(w3.shape), full(b3.shape),
            full(g3.shape), full(be3.shape), full(bias.shape),
        ],
        out_specs=pl.BlockSpec((bs,), lambda i: (i,)),
        out_shape=jax.ShapeDtypeStruct((B,), jnp.float32),
    )(emb, x2, fsum, s, w1a, w1b, b1, g1, be1, w2, b2, g2, be2,
      w3, b3, g3, be3, bias)


def kernel(train_x1, train_x2, fm_first_w, fm_second_w, bias,
           W1, b1, g1, be1, W2, b2, g2, be2, W3, b3, g3, be3):
    idx = train_x1.reshape(N_IDX // K, K)
    tq = fm_second_w.T.reshape(2, 8, TCOL, 128).transpose(0, 2, 1, 3)
    t2 = _sc_transpose(tq)
    emb_flat, fsum = _sc_gather(idx, t2, fm_first_w.T.reshape(V))
    emb = emb_flat.reshape(B, D)
    s = jnp.tile(jnp.eye(EMB, dtype=jnp.float32), (FIELD, 1))
    return _mlp(emb, train_x2, fsum, s, W1[:D], W1[D:], b1, g1, be1,
                W2, b2, g2, be2, W3, b3, g3, be3, bias)


# speedup vs baseline: 2.8288x; 1.7226x over previous
import functools

import jax
import jax.numpy as jnp
from jax import lax
from jax.experimental import pallas as pl
from jax.experimental.pallas import tpu as pltpu
from jax.experimental.pallas import tpu_sc as plsc

B = 16384
FIELD = 26
EMB = 16
V = 26 * 40000
D = FIELD * EMB
NUMF = 13
EPS = 1e-5

NC, NS = 2, 16
NW = NC * NS
N_IDX = B * FIELD
PER_W = N_IDX // NW
K = 128
NSTREAM = PER_W // K

NB = 4
MB = NB * K
NMACRO = NSTREAM // NB
SAMP_W = B // NW

TCOL = V // 128
CTC = 13
CV = CTC * 128
NCHUNK = TCOL // CTC
NCHUNK_LO = NCHUNK // NW
NCHUNK_REM = NCHUNK - NCHUNK_LO * NW

_sc_mesh = plsc.VectorSubcoreMesh(core_axis_name="c", subcore_axis_name="s")


@functools.partial(
    pl.kernel,
    out_type=jax.ShapeDtypeStruct((V * EMB,), jnp.float32),
    mesh=_sc_mesh,
    compiler_params=pltpu.CompilerParams(use_tc_tiling_on_sc=False,
                                        needs_layout_passes=False),
    scratch_types=[
        pltpu.VMEM((2, 2, CTC, 8, 128), jnp.float32),
        pltpu.VMEM((2, CV * EMB), jnp.float32),
        pltpu.SemaphoreType.DMA,
        pltpu.SemaphoreType.DMA,
        pltpu.SemaphoreType.DMA,
        pltpu.SemaphoreType.DMA,
    ],
)
def _sc_transpose(tq_hbm, t2_out, in4, out2,
                  isem0, isem1, osem0, osem1):
    wid = lax.axis_index("s") * NC + lax.axis_index("c")
    cnt = jnp.where(wid < NCHUNK_REM, NCHUNK_LO + 1, NCHUNK_LO)
    isem = (isem0, isem1)
    osem = (osem0, osem1)
    st16 = lax.iota(jnp.int32, 16) * EMB

    def fetch(i, p):
        c = wid + NW * i
        for tr in range(2):
            pltpu.async_copy(tq_hbm.at[tr].at[pl.ds(c * CTC, CTC)],
                             in4.at[p].at[tr], isem[p])

    def wait_fetch(i, p):
        c = wid + NW * i
        for tr in range(2):
            pltpu.make_async_copy(tq_hbm.at[tr].at[pl.ds(c * CTC, CTC)],
                                  in4.at[p].at[tr], isem[p]).wait()

    def wait_write(i, p):
        c = wid + NW * i
        pltpu.make_async_copy(out2.at[p],
                              t2_out.at[pl.ds(c * CV * EMB, CV * EMB)],
                              osem[p]).wait()

    fetch(0, 0)

    def chunk_body(i, p):
        @pl.when(i + 1 < cnt)
        def _():
            fetch(i + 1, 1 - p)
        wait_fetch(i, p)
        src = in4.at[p]
        dst = out2.at[p]

        def interleave(tcl, carry):
            cbase = tcl * (128 * EMB)
            for d in range(EMB):
                row = src.at[d // 8].at[tcl].at[d % 8]
                for vv0 in range(0, 128, 16):
                    vec = row[pl.ds(vv0, 16)]
                    idx = st16 + (cbase + vv0 * EMB + d)
                    plsc.store_scatter(dst, [idx], vec)
            return carry

        lax.fori_loop(0, CTC, interleave, 0)
        c = wid + NW * i
        pltpu.async_copy(out2.at[p],
                         t2_out.at[pl.ds(c * CV * EMB, CV * EMB)], osem[p])

    def body(j, carry):
        i = 2 * j

        @pl.when(i < cnt)
        def _():
            @pl.when(i >= 2)
            def _():
                wait_write(i - 2, 0)
            chunk_body(i, 0)

        @pl.when(i + 1 < cnt)
        def _():
            @pl.when(i >= 1)
            def _():
                wait_write(i - 1, 1)
            chunk_body(i + 1, 1)

        return carry

    lax.fori_loop(0, (NCHUNK_LO + 2) // 2, body, 0)

    hi = NCHUNK_LO + 1

    @pl.when(cnt == hi)
    def _():
        wait_write(hi - 2, (hi - 2) % 2)
        wait_write(hi - 1, (hi - 1) % 2)

    @pl.when(cnt == hi - 1)
    def _():
        wait_write(hi - 3, (hi - 3) % 2)
        wait_write(hi - 2, (hi - 2) % 2)


@functools.partial(
    pl.kernel,
    out_type=(
        jax.ShapeDtypeStruct((N_IDX, EMB), jnp.float32),
        jax.ShapeDtypeStruct((B,), jnp.float32),
    ),
    mesh=_sc_mesh,
    compiler_params=pltpu.CompilerParams(use_tc_tiling_on_sc=False,
                                        needs_layout_passes=False),
    scratch_types=[
        pltpu.VMEM((NSTREAM, K), jnp.int32),
        pltpu.VMEM((2, MB, EMB), jnp.float32),
        pltpu.VMEM((PER_W,), jnp.float32),
        pltpu.VMEM((SAMP_W,), jnp.float32),
        pltpu.SemaphoreType.DMA,
        pltpu.SemaphoreType.DMA,
        pltpu.SemaphoreType.DMA,
    ],
)
def _sc_gather(idx_hbm, second_hbm, first_hbm, emb_out, fsum_out,
               idx_v, rows2, fst_v, fsum_v, rsem0, rsem1, fsem):
    wid = lax.axis_index("s") * NC + lax.axis_index("c")
    row0 = wid * NSTREAM
    base = wid * PER_W
    rsem = (rsem0, rsem1)
    pltpu.sync_copy(idx_hbm.at[pl.ds(row0, NSTREAM)], idx_v)

    def fire(m, p):
        for b in range(NB):
            j = m * NB + b
            pltpu.async_copy(second_hbm.at[idx_v.at[j]],
                             rows2.at[p].at[pl.ds(b * K, K)], rsem[p])
            pltpu.async_copy(first_hbm.at[idx_v.at[j]],
                             fst_v.at[pl.ds((m * NB + b) * K, K)], fsem)

    def drain(m, p):
        for b in range(NB):
            j = m * NB + b
            pltpu.make_async_copy(second_hbm.at[idx_v.at[j]],
                                  rows2.at[p].at[pl.ds(b * K, K)],
                                  rsem[p]).wait()

    def write(m, p):
        pltpu.sync_copy(rows2.at[p], emb_out.at[pl.ds(base + m * MB, MB)])

    fire(0, 0)

    def body(i, carry):
        mm = 2 * i
        fire(mm + 1, 1)
        drain(mm, 0)
        write(mm, 0)

        @pl.when(i < NMACRO // 2 - 1)
        def _():
            fire(mm + 2, 0)

        drain(mm + 1, 1)
        write(mm + 1, 1)
        return carry

    lax.fori_loop(0, NMACRO // 2, body, 0)

    def fdrain(j, carry):
        pltpu.make_async_copy(first_hbm.at[idx_v.at[j]],
                              fst_v.at[pl.ds(j * K, K)], fsem).wait()
        return carry

    lax.fori_loop(0, NSTREAM, fdrain, 0)

    iota16 = lax.iota(jnp.int32, 16)

    def fsum_body(g, carry):
        acc = jnp.zeros((16,), jnp.float32)
        for f in range(FIELD):
            ridx = (g * 16 + iota16) * FIELD + f
            acc = acc + plsc.load_gather(fst_v, [ridx])
        fsum_v[pl.ds(g * 16, 16)] = acc
        return carry

    lax.fori_loop(0, SAMP_W // 16, fsum_body, 0)
    pltpu.sync_copy(fsum_v, fsum_out.at[pl.ds(wid * SAMP_W, SAMP_W)])


def _mlp_body(emb_ref, x2_ref, fsum_ref, s_ref,
              w1a_ref, w1b_ref, b1_ref, g1_ref, be1_ref,
              w2_ref, b2_ref, g2_ref, be2_ref,
              w3_ref, b3_ref, g3_ref, be3_ref, bias_ref, out_ref):
    x = emb_ref[...]
    inv = 1.0 / jnp.sqrt(1.0 + EPS)
    sumvec = jnp.dot(x, s_ref[...], preferred_element_type=jnp.float32)
    fm2 = 0.5 * (jnp.sum(sumvec * sumvec, axis=1) - jnp.sum(x * x, axis=1))
    h = (jnp.dot(x, w1a_ref[...], preferred_element_type=jnp.float32)
         + jnp.dot(x2_ref[...], w1b_ref[...], preferred_element_type=jnp.float32)
         + b1_ref[...])
    h = jnp.maximum(h * (g1_ref[...] * inv) + be1_ref[...], 0.0)
    h = jnp.dot(h, w2_ref[...], preferred_element_type=jnp.float32) + b2_ref[...]
    h = jnp.maximum(h * (g2_ref[...] * inv) + be2_ref[...], 0.0)
    h = jnp.dot(h, w3_ref[...], preferred_element_type=jnp.float32) + b3_ref[...]
    h = jnp.maximum(h * (g3_ref[...] * inv) + be3_ref[...], 0.0)
    logit = fsum_ref[...] + fm2 + jnp.sum(h, axis=1) + bias_ref[0]
    out_ref[...] = jax.nn.sigmoid(logit)


def _mlp(emb, x2, fsum, s, w1a, w1b, b1, g1, be1, w2, b2, g2, be2,
         w3, b3, g3, be3, bias):
    bs = 2048
    grid = (B // bs,)
    full = lambda shape: pl.BlockSpec(shape, lambda i: tuple(0 for _ in shape))
    return pl.pallas_call(
        _mlp_body,
        grid=grid,
        in_specs=[
            pl.BlockSpec((bs, D), lambda i: (i, 0)),
            pl.BlockSpec((bs, NUMF), lambda i: (i, 0)),
            pl.BlockSpec((bs,), lambda i: (i,)),
            full(s.shape), full(w1a.shape), full(w1b.shape), full(b1.shape),
            full(g1.shape), full(be1.shape), full(w2.shape), full(b2.shape),
            full(g2.shape), full(be2.shape), full(w3.shape), full(b3.shape),
            full(g3.shape), full(be3.shape), full(bias.shape),
        ],
        out_specs=pl.BlockSpec((bs,), lambda i: (i,)),
        out_shape=jax.ShapeDtypeStruct((B,), jnp.float32),
    )(emb, x2, fsum, s, w1a, w1b, b1, g1, be1, w2, b2, g2, be2,
      w3, b3, g3, be3, bias)


def kernel(train_x1, train_x2, fm_first_w, fm_second_w, bias,
           W1, b1, g1, be1, W2, b2, g2, be2, W3, b3, g3, be3):
    idx = train_x1.reshape(N_IDX // K, K)
    tq = fm_second_w.T.reshape(2, 8, TCOL, 128).transpose(0, 2, 1, 3)
    t2 = _sc_transpose(tq).reshape(V, EMB)
    emb_flat, fsum = _sc_gather(idx, t2, fm_first_w.T.reshape(V))
    emb = emb_flat.reshape(B, D)
    s = jnp.tile(jnp.eye(EMB, dtype=jnp.float32), (FIELD, 1))
    return _mlp(emb, train_x2, fsum, s, W1[:D], W1[D:], b1, g1, be1,
                W2, b2, g2, be2, W3, b3, g3, be3, bias)
